# baseline (speedup 1.0000x reference)
import functools

import jax
import jax.numpy as jnp
from jax import lax
from jax.experimental import pallas as pl
from jax.experimental.pallas import tpu as pltpu
from jax.experimental.pallas import tpu_sc as plsc

E = 16
SCALE = 0.5
NEG = -1e30
TBLK = 512
BLK = 256
T = 4096
D = 1024
B = 64
NBLK = (2 * T + E * (BLK - 1) + BLK - 1) // BLK
CAP = NBLK * BLK


def _top2(logits):
    lane = lax.broadcasted_iota(jnp.int32, logits.shape, 1)
    m1 = jnp.max(logits, axis=1, keepdims=True)
    i1 = jnp.min(jnp.where(logits == m1, lane, E), axis=1, keepdims=True)
    sel1 = lane == i1
    l2 = jnp.where(sel1, NEG, logits)
    m2 = jnp.max(l2, axis=1, keepdims=True)
    i2 = jnp.min(jnp.where(l2 == m2, lane, E), axis=1, keepdims=True)
    sel2 = lane == i2
    g1 = 1.0 / (1.0 + jnp.exp(m2 - m1))
    return i1, i2, sel1, sel2, g1[:, 0], 1.0 - g1[:, 0]


def _k1_body(x_ref, wg_ref, i1_ref, i2_ref, r1_ref, r2_ref, g1_ref, g2_ref,
             cnt_ref, cnt_s):
    i = pl.program_id(0)
    logits = jnp.dot(x_ref[...], wg_ref[...], preferred_element_type=jnp.float32)
    i1, i2, sel1, sel2, g1, g2 = _top2(logits)

    @pl.when(i == 0)
    def _():
        cnt_s[...] = jnp.zeros_like(cnt_s)

    cnt0 = cnt_s[...]
    asgn = sel1.astype(jnp.float32) + sel2.astype(jnp.float32)
    r_io = lax.broadcasted_iota(jnp.int32, (TBLK, TBLK), 0)
    c_io = lax.broadcasted_iota(jnp.int32, (TBLK, TBLK), 1)
    ltri = (c_io < r_io).astype(jnp.float32)
    ranks = jnp.dot(ltri, asgn, preferred_element_type=jnp.float32) + cnt0
    r1 = jnp.sum(jnp.where(sel1, ranks, 0.0), axis=1)
    r2 = jnp.sum(jnp.where(sel2, ranks, 0.0), axis=1)
    cnt1 = cnt0 + jnp.sum(asgn, axis=0, keepdims=True)
    cnt_s[...] = cnt1

    i1_ref[0, 0, :] = i1[:, 0]
    i2_ref[0, 0, :] = i2[:, 0]
    r1_ref[0, 0, :] = r1.astype(jnp.int32)
    r2_ref[0, 0, :] = r2.astype(jnp.int32)
    g1_ref[0, 0, :] = g1
    g2_ref[0, 0, :] = g2
    cnt_ref[...] = jnp.broadcast_to(cnt1, cnt_ref.shape).astype(jnp.int32)


def _k2_body(i1_ref, i2_ref, r1_ref, r2_ref, g1_ref, g2_ref, cnt_ref,
             p1_ref, p2_ref, g1r_ref, g2r_ref, be_ref):
    cnt = cnt_ref[0:1, :]
    nb_e = (cnt + (BLK - 1)) // BLK
    cnt_pad = (nb_e * BLK).astype(jnp.float32)
    r_io = lax.broadcasted_iota(jnp.int32, (E, E), 0)
    c_io = lax.broadcasted_iota(jnp.int32, (E, E), 1)
    utri = (r_io < c_io).astype(jnp.float32)
    cnt_pad8 = jnp.broadcast_to(cnt_pad, (8, E))
    rowoff = jnp.dot(cnt_pad8, utri, preferred_element_type=jnp.float32)[0:1, :]
    rowoff_i = rowoff.astype(jnp.int32)
    bend = (rowoff_i + cnt_pad.astype(jnp.int32)) // BLK
    b_io = lax.broadcasted_iota(jnp.int32, (128, E), 0)
    be = jnp.sum((b_io >= bend).astype(jnp.int32), axis=1)
    be_ref[...] = jnp.minimum(be, E - 1)

    lane = lax.broadcasted_iota(jnp.int32, (TBLK, E), 1)
    i1 = i1_ref[0, 0, :]
    i2 = i2_ref[0, 0, :]
    oh1 = lane == i1[:, None]
    oh2 = lane == i2[:, None]
    off1 = jnp.sum(jnp.where(oh1, rowoff_i, 0), axis=1)
    off2 = jnp.sum(jnp.where(oh2, rowoff_i, 0), axis=1)
    p1_ref[0, 0, :] = off1 + r1_ref[0, 0, :]
    p2_ref[0, 0, :] = off2 + r2_ref[0, 0, :]
    g1r_ref[...] = jnp.broadcast_to(g1_ref[0, 0, :][:, None], (TBLK, E))
    g2r_ref[...] = jnp.broadcast_to(g2_ref[0, 0, :][:, None], (TBLK, E))


def _routing(x, w_gate):
    nb = T // TBLK
    tb3 = lambda dt: jax.ShapeDtypeStruct((nb, 1, TBLK), dt)
    spec3 = pl.BlockSpec((1, 1, TBLK), lambda i: (i, 0, 0))
    i1, i2, r1, r2, g1, g2, cnt = pl.pallas_call(
        _k1_body,
        grid=(nb,),
        in_specs=[pl.BlockSpec((TBLK, D), lambda i: (i, 0)),
                  pl.BlockSpec((D, E), lambda i: (0, 0))],
        out_specs=[spec3] * 6 + [pl.BlockSpec((8, E), lambda i: (0, 0))],
        out_shape=[tb3(jnp.int32)] * 4 + [tb3(jnp.float32)] * 2
                  + [jax.ShapeDtypeStruct((8, E), jnp.int32)],
        scratch_shapes=[pltpu.VMEM((1, E), jnp.float32)],
    )(x, w_gate)

    p1, p2, g1r, g2r, be = pl.pallas_call(
        _k2_body,
        grid=(nb,),
        in_specs=[spec3] * 6 + [pl.BlockSpec((8, E), lambda i: (0, 0))],
        out_specs=[spec3, spec3,
                   pl.BlockSpec((TBLK, E), lambda i: (i, 0)),
                   pl.BlockSpec((TBLK, E), lambda i: (i, 0)),
                   pl.BlockSpec((128,), lambda i: (0,))],
        out_shape=[tb3(jnp.int32)] * 2
                  + [jax.ShapeDtypeStruct((T, E), jnp.float32)] * 2
                  + [jax.ShapeDtypeStruct((128,), jnp.int32)],
    )(i1, i2, r1, r2, g1, g2, cnt)
    return (jnp.reshape(p1, (T // 16, 16)), jnp.reshape(p2, (T // 16, 16)),
            g1r, g2r, be[:NBLK])


def _k3_body(be_ref, xg_ref, dw_ref, db_ref, uw_ref, ub_ref, y_ref):
    h = jnp.dot(xg_ref[...], dw_ref[0], preferred_element_type=jnp.float32)
    h = jnp.maximum(h + db_ref[0], 0.0)
    y = jnp.dot(h.astype(jnp.bfloat16), uw_ref[0],
                preferred_element_type=jnp.float32)
    y_ref[...] = (y + ub_ref[0]) * SCALE


def _grouped_matmul(be, xg, dw_bf, down_b, uw_bf, up_b):
    return pl.pallas_call(
        _k3_body,
        grid_spec=pltpu.PrefetchScalarGridSpec(
            num_scalar_prefetch=1,
            grid=(NBLK,),
            in_specs=[
                pl.BlockSpec((BLK, D), lambda i, s: (i, 0)),
                pl.BlockSpec((1, D, B), lambda i, s: (s[i], 0, 0)),
                pl.BlockSpec((1, 1, B), lambda i, s: (s[i], 0, 0)),
                pl.BlockSpec((1, B, D), lambda i, s: (s[i], 0, 0)),
                pl.BlockSpec((1, 1, D), lambda i, s: (s[i], 0, 0)),
            ],
            out_specs=pl.BlockSpec((BLK, D), lambda i, s: (i, 0)),
        ),
        out_shape=jax.ShapeDtypeStruct((CAP, D), jnp.float32),
    )(be, xg, dw_bf, down_b[:, None, :], uw_bf, up_b[:, None, :])


def _sc_mesh():
    info = plsc.get_sparse_core_info()
    return (plsc.VectorSubcoreMesh(core_axis_name="c", subcore_axis_name="s"),
            info.num_cores, info.num_subcores)


def _dispatch(x_f2, p1, p2):
    D2 = D // 2
    mesh, nc, ns = _sc_mesh()
    tpw = T // (nc * ns)
    nch = tpw // 16

    @functools.partial(
        pl.kernel,
        out_type=jax.ShapeDtypeStruct((CAP, D2), jnp.float32),
        mesh=mesh,
        scratch_types=[pltpu.VMEM((nch, 16), jnp.int32),
                       pltpu.VMEM((nch, 16), jnp.int32),
                       pltpu.VMEM((2, 16, D2), jnp.float32),
                       pltpu.SemaphoreType.DMA,
                       pltpu.SemaphoreType.DMA,
                       pltpu.SemaphoreType.DMA,
                       pltpu.SemaphoreType.DMA],
    )
    def k(x_hbm, p1_hbm, p2_hbm, xg_hbm, i1_v, i2_v, bufs,
          lsem0, lsem1, ssem0, ssem1):
        wid = lax.axis_index("s") * nc + lax.axis_index("c")
        tbase = wid * tpw
        lsem = (lsem0, lsem1)
        ssem = (ssem0, ssem1)
        pltpu.sync_copy(p1_hbm.at[pl.ds(wid * nch, nch)], i1_v)
        pltpu.sync_copy(p2_hbm.at[pl.ds(wid * nch, nch)], i2_v)

        def load(j):
            return pltpu.async_copy(
                x_hbm.at[pl.ds(tbase + j * 16, 16)], bufs.at[j % 2],
                lsem[j % 2])

        lh = {0: load(0)}
        sh = {0: [], 1: []}
        for j in range(nch):
            if j + 1 < nch:
                for h in sh[(j + 1) % 2]:
                    h.wait()
                sh[(j + 1) % 2] = []
                lh[(j + 1) % 2] = load(j + 1)
            lh[j % 2].wait()
            sh[j % 2] = [
                pltpu.async_copy(bufs.at[j % 2], xg_hbm.at[i1_v.at[j]],
                                 ssem[j % 2]),
                pltpu.async_copy(bufs.at[j % 2], xg_hbm.at[i2_v.at[j]],
                                 ssem[j % 2]),
            ]
        for lst in sh.values():
            for h in lst:
                h.wait()

    return k(x_f2, p1, p2)


def _combine(y, p1, p2, g1r, g2r):
    mesh, nc, ns = _sc_mesh()
    tpw = T // (nc * ns)
    nch = tpw // 16

    @functools.partial(
        pl.kernel,
        out_type=jax.ShapeDtypeStruct((T, D), jnp.float32),
        mesh=mesh,
        scratch_types=[pltpu.VMEM((nch, 16), jnp.int32),
                       pltpu.VMEM((nch, 16), jnp.int32),
                       pltpu.VMEM((2, 16, D), jnp.float32),
                       pltpu.VMEM((2, 16, D), jnp.float32),
                       pltpu.VMEM((tpw, E), jnp.float32),
                       pltpu.VMEM((tpw, E), jnp.float32),
                       pltpu.SemaphoreType.DMA,
                       pltpu.SemaphoreType.DMA,
                       pltpu.SemaphoreType.DMA,
                       pltpu.SemaphoreType.DMA],
    )
    def k(y_hbm, p1_hbm, p2_hbm, g1_hbm, g2_hbm, out_hbm,
          i1_v, i2_v, bufa, bufb, ga_v, gb_v,
          gsem0, gsem1, osem0, osem1):
        wid = lax.axis_index("s") * nc + lax.axis_index("c")
        tbase = wid * tpw
        gsem = (gsem0, gsem1)
        osem = (osem0, osem1)
        pltpu.sync_copy(p1_hbm.at[pl.ds(wid * nch, nch)], i1_v)
        pltpu.sync_copy(p2_hbm.at[pl.ds(wid * nch, nch)], i2_v)
        pltpu.sync_copy(g1_hbm.at[pl.ds(tbase, tpw)], ga_v)
        pltpu.sync_copy(g2_hbm.at[pl.ds(tbase, tpw)], gb_v)

        def gath(j):
            return [
                pltpu.async_copy(y_hbm.at[i1_v.at[j]], bufa.at[j % 2],
                                 gsem[j % 2]),
                pltpu.async_copy(y_hbm.at[i2_v.at[j]], bufb.at[j % 2],
                                 gsem[j % 2]),
            ]

        gh = {0: gath(0), 1: []}
        oh = {0: None, 1: None}
        for j in range(nch):
            if j + 1 < nch:
                if oh[(j + 1) % 2] is not None:
                    oh[(j + 1) % 2].wait()
                    oh[(j + 1) % 2] = None
                gh[(j + 1) % 2] = gath(j + 1)
            for h in gh[j % 2]:
                h.wait()
            for t in range(16):
                ga = ga_v[j * 16 + t, :]
                gb = gb_v[j * 16 + t, :]

                def col(c, _):
                    cs = pl.ds(c * 16, 16)
                    bufa[j % 2, t, cs] = (ga * bufa[j % 2, t, cs]
                                          + gb * bufb[j % 2, t, cs])
                    return 0

                lax.fori_loop(0, D // 16, col, 0, unroll=8)
            oh[j % 2] = pltpu.async_copy(
                bufa.at[j % 2], out_hbm.at[pl.ds(tbase + j * 16, 16)],
                osem[j % 2])
        for h in oh.values():
            if h is not None:
                h.wait()

    return k(y, p1, p2, g1r, g2r)


def _pack_words(a):
    return lax.bitcast_convert_type(
        a.reshape(*a.shape[:-1], a.shape[-1] // 2, 2), jnp.float32)


def _unpack_words(a):
    b = lax.bitcast_convert_type(a, jnp.bfloat16)
    return b.reshape(*a.shape[:-1], a.shape[-1] * 2)


@jax.jit
def kernel(x, w_gate, w_noise, down_w, down_b, up_w, up_b):
    del w_noise
    p1, p2, g1r, g2r, be = _routing(x, w_gate)
    xg_f2 = _dispatch(_pack_words(x.astype(jnp.bfloat16)), p1, p2)
    y = _grouped_matmul(be, _unpack_words(xg_f2), down_w.astype(jnp.bfloat16),
                        down_b, up_w.astype(jnp.bfloat16), up_b)
    return _combine(y, p1, p2, g1r, g2r)


# speedup vs baseline: 4.3506x; 4.3506x over previous
import functools

import jax
import jax.numpy as jnp
from jax.experimental import pallas as pl

E = 16
TOPK = 2
SCALE = 0.5
NEG = -1e30


def _gates_dense(logits):
    lane = jax.lax.broadcasted_iota(jnp.int32, logits.shape, 1)
    m1 = jnp.max(logits, axis=1, keepdims=True)
    i1 = jnp.min(jnp.where(logits == m1, lane, E), axis=1, keepdims=True)
    sel1 = lane == i1
    l2 = jnp.where(sel1, NEG, logits)
    m2 = jnp.max(l2, axis=1, keepdims=True)
    i2 = jnp.min(jnp.where(l2 == m2, lane, E), axis=1, keepdims=True)
    sel2 = lane == i2
    e21 = jnp.exp(m2 - m1)
    g1 = 1.0 / (1.0 + e21)
    g2 = 1.0 - g1
    return jnp.where(sel1, g1, 0.0) + jnp.where(sel2, g2, 0.0)


def _dense_body(x_ref, wg_ref, dw_ref, db_ref, uw_ref, ub_ref, o_ref):
    xb = x_ref[...]
    logits = jnp.dot(xb, wg_ref[...], preferred_element_type=jnp.float32)
    gates = _gates_dense(logits)
    blk, d = xb.shape
    xb_bf = xb.astype(jnp.bfloat16)
    acc = jnp.zeros((blk, d), jnp.float32)
    for e in range(E):
        h = jnp.dot(xb_bf, dw_ref[e], preferred_element_type=jnp.float32)
        h = jnp.maximum(h + db_ref[e][None, :], 0.0)
        y = jnp.dot(h.astype(jnp.bfloat16), uw_ref[e],
                    preferred_element_type=jnp.float32)
        y = y + ub_ref[e][None, :]
        acc = acc + gates[:, e][:, None] * y
    o_ref[...] = acc * SCALE


@functools.partial(jax.jit, static_argnames=("interpret",))
def kernel(x, w_gate, w_noise, down_w, down_b, up_w, up_b, interpret=False):
    del w_noise
    t, d = x.shape
    blk = 256
    b = down_w.shape[-1]
    full = lambda shape: pl.BlockSpec(shape, lambda i: tuple(0 for _ in shape))
    return pl.pallas_call(
        _dense_body,
        grid=(t // blk,),
        in_specs=[
            pl.BlockSpec((blk, d), lambda i: (i, 0)),
            full((d, E)),
            full((E, d, b)),
            full((E, b)),
            full((E, b, d)),
            full((E, d)),
        ],
        out_specs=pl.BlockSpec((blk, d), lambda i: (i, 0)),
        out_shape=jax.ShapeDtypeStruct((t, d), jnp.float32),
        interpret=interpret,
    )(x, w_gate, down_w.astype(jnp.bfloat16), down_b,
      up_w.astype(jnp.bfloat16), up_b)


# speedup vs baseline: 5.3350x; 1.2263x over previous
import functools

import jax
import jax.numpy as jnp
from jax.experimental import pallas as pl

E = 16
TOPK = 2
SCALE = 0.5
NEG = -1e30


def _gates_dense(logits):
    lane = jax.lax.broadcasted_iota(jnp.int32, logits.shape, 1)
    m1 = jnp.max(logits, axis=1, keepdims=True)
    i1 = jnp.min(jnp.where(logits == m1, lane, E), axis=1, keepdims=True)
    sel1 = lane == i1
    l2 = jnp.where(sel1, NEG, logits)
    m2 = jnp.max(l2, axis=1, keepdims=True)
    i2 = jnp.min(jnp.where(l2 == m2, lane, E), axis=1, keepdims=True)
    sel2 = lane == i2
    e21 = jnp.exp(m2 - m1)
    g1 = 1.0 / (1.0 + e21)
    g2 = 1.0 - g1
    return jnp.where(sel1, g1, 0.0) + jnp.where(sel2, g2, 0.0)


def _dense_body(x_ref, wg_ref, dw_ref, db_ref, uw_ref, ub_ref, o_ref):
    xb = x_ref[...]
    logits = jnp.dot(xb, wg_ref[...], preferred_element_type=jnp.float32)
    gates = _gates_dense(logits)
    blk, d = xb.shape
    acc = jnp.zeros((blk, d), jnp.float32)
    for e in range(E):
        h = jnp.dot(xb, dw_ref[e], preferred_element_type=jnp.float32)
        h = jnp.maximum(h + db_ref[e][None, :], 0.0)
        y = jnp.dot(h, uw_ref[e], preferred_element_type=jnp.float32)
        y = y + ub_ref[e][None, :]
        acc = acc + gates[:, e][:, None] * y
    o_ref[...] = acc * SCALE


@functools.partial(jax.jit, static_argnames=("interpret",))
def kernel(x, w_gate, w_noise, down_w, down_b, up_w, up_b, interpret=False):
    del w_noise
    t, d = x.shape
    blk = 256
    b = down_w.shape[-1]
    full = lambda shape: pl.BlockSpec(shape, lambda i: tuple(0 for _ in shape))
    return pl.pallas_call(
        _dense_body,
        grid=(t // blk,),
        in_specs=[
            pl.BlockSpec((blk, d), lambda i: (i, 0)),
            full((d, E)),
            full((E, d, b)),
            full((E, b)),
            full((E, b, d)),
            full((E, d)),
        ],
        out_specs=pl.BlockSpec((blk, d), lambda i: (i, 0)),
        out_shape=jax.ShapeDtypeStruct((t, d), jnp.float32),
        interpret=interpret,
    )(x, w_gate, down_w, down_b, up_w, up_b)
